# baseline (speedup 1.0000x reference)
import functools

import jax
import jax.numpy as jnp
from jax import lax
from jax.experimental import pallas as pl
from jax.experimental.pallas import tpu as pltpu
from jax.experimental.pallas import tpu_sc as plsc

N = 10000
E = 320000
B = 256
LIB = 1000
D = 128

NC = 2
NS = 16
NW = NC * NS
L = 16

SLAB = 312
SLAB_LAST = SLAB + (N - SLAB * NW)
PSLAB = 320
PSLAB_LAST = N - PSLAB * (NW - 1)
EPW = E // NW
EK = 128
ESTEPS = EPW // EK
RPW = 624
RPW_LAST = N - RPW * (NS - 1)

_MESH = plsc.VectorSubcoreMesh(core_axis_name="c", subcore_axis_name="s")


def _worker_id():
    return lax.axis_index("s") * NC + lax.axis_index("c")


def _emb_gather_body(xv_hbm, emb_hbm, out_hbm, idx_v, rows_v, idx2_v, rows2_v, sem):
    w = _worker_id()
    base = w * SLAB
    for i in range(3):
        off = base + i * 104
        pltpu.sync_copy(xv_hbm.at[pl.ds(off, 104)], idx_v)
        pltpu.async_copy(emb_hbm.at[idx_v], rows_v, sem).wait()
        pltpu.sync_copy(rows_v, out_hbm.at[pl.ds(off, 104)])

    @pl.when(w == NW - 1)
    def _():
        off = base + SLAB
        pltpu.sync_copy(xv_hbm.at[pl.ds(off, 16)], idx2_v)
        pltpu.async_copy(emb_hbm.at[idx2_v], rows2_v, sem).wait()
        pltpu.sync_copy(rows2_v, out_hbm.at[pl.ds(off, 16)])


@jax.jit
def _emb_gather(xv, emb):
    return pl.kernel(
        _emb_gather_body,
        out_type=jax.ShapeDtypeStruct((N, D), jnp.float32),
        mesh=_MESH,
        scratch_types=[
            pltpu.VMEM((104,), jnp.int32),
            pltpu.VMEM((104, D), jnp.float32),
            pltpu.VMEM((16,), jnp.int32),
            pltpu.VMEM((16, D), jnp.float32),
            pltpu.SemaphoreType.DMA,
        ],
    )(xv, emb)


ECHUNK = 1600
TRASH = 335
TRASH_REC = TRASH << 14
CAPW = E + EK
VBUF = 2176
FLUSH = 2048


def _prefix16(m):
    iota = lax.iota(jnp.int32, L)
    for k in (1, 2, 4, 8):
        g = m.at[jnp.maximum(iota - k, 0)].get(mode="promise_in_bounds")
        m = m + jnp.where(iota >= k, g, 0)
    return m


def _bucket_body(src_hbm, dst_hbm, rec_hbm, cnt_hbm, svbuf, dvbuf, vbuf, cbuf, sem):
    w = _worker_id()
    low = w * SLAB
    hi = low + jnp.where(w == NW - 1, SLAB_LAST, SLAB)
    full_mask = jnp.ones((L,), jnp.bool_)
    tvec = jnp.full((L,), TRASH_REC, jnp.int32)

    def outer(oi, carry):
        pltpu.sync_copy(src_hbm.at[pl.ds(oi * ECHUNK, ECHUNK)], svbuf)
        pltpu.sync_copy(dst_hbm.at[pl.ds(oi * ECHUNK, ECHUNK)], dvbuf)

        def inner(gk, c2):
            vcur, gcur = c2
            recs, masks, cms = [], [], []
            for u in range(4):
                sv = svbuf[pl.ds((gk * 4 + u) * L, L)]
                dv = dvbuf[pl.ds((gk * 4 + u) * L, L)]
                mask = (dv >= low) & (dv < hi)
                rec = sv | ((dv - low) << 14)
                cm = _prefix16(jnp.where(mask, 1, 0))
                recs.append(rec)
                masks.append(mask)
                cms.append(cm)
            base = vcur
            for u in range(4):
                plsc.store_scatter(vbuf, [base + cms[u] - 1], recs[u], mask=masks[u])
                base = base + cms[u][L - 1]
            vcur = base

            def do_flush(args):
                vc, gc = args
                pltpu.sync_copy(vbuf.at[pl.ds(0, FLUSH)], rec_hbm.at[pl.ds(pl.multiple_of(w * CAPW + gc, 2048), FLUSH)])
                for t in range(4):
                    tail = vbuf[pl.ds(FLUSH + t * L, L)]
                    vbuf[pl.ds(t * L, L)] = tail
                return (vc - FLUSH, gc + FLUSH)

            return lax.cond(vcur >= FLUSH, do_flush, lambda a: a, (vcur, gcur))

        return lax.fori_loop(0, ECHUNK // L // 4, inner, carry)

    vcur, gcur = lax.fori_loop(0, E // ECHUNK, outer, (jnp.int32(0), jnp.int32(0)))

    plsc.store_scatter(vbuf, [vcur + lax.iota(jnp.int32, L)], tvec, mask=full_mask)
    vcur = ((vcur + L - 1) // L) * L

    def pad_more(args):
        vc, gc = args
        plsc.store_scatter(vbuf, [vc + lax.iota(jnp.int32, L)], tvec, mask=full_mask)
        return (vc + L, gc)

    def pad_cond(args):
        vc, gc = args
        return ((gc + vc) % EK) != 0

    vcur, gcur = lax.while_loop(pad_cond, pad_more, (vcur, gcur))

    def fl(i, args):
        vc, gc = args
        pltpu.sync_copy(vbuf.at[pl.ds(i * L, L)], rec_hbm.at[pl.ds(pl.multiple_of(w * CAPW + gc + i * L, L), L)])
        return (vc, gc)

    lax.fori_loop(0, vcur // L, fl, (vcur, gcur))
    total = gcur + vcur

    cbuf[...] = jnp.where(lax.iota(jnp.int32, L) == 0, total, 0)
    pltpu.sync_copy(cbuf, cnt_hbm.at[pl.ds(w * L, L)])


@jax.jit
def _bucket(src, dst):
    return pl.kernel(
        _bucket_body,
        out_type=(
            jax.ShapeDtypeStruct((NW * CAPW,), jnp.int32),
            jax.ShapeDtypeStruct((NW * L,), jnp.int32),
        ),
        mesh=_MESH,
        scratch_types=[
            pltpu.VMEM((ECHUNK,), jnp.int32),
            pltpu.VMEM((ECHUNK,), jnp.int32),
            pltpu.VMEM((VBUF,), jnp.int32),
            pltpu.VMEM((L,), jnp.int32),
            pltpu.SemaphoreType.DMA,
        ],
        compiler_params=pltpu.CompilerParams(needs_layout_passes=False),
    )(src, dst)


RECCAP = 16384


def _edge_agg_body(h_hbm, rec_hbm, cnt_hbm, out_hbm, recbig, sidxA, sidxB,
                   rowsA, rowsB, acc, cbuf, semA, semB):
    w = _worker_id()
    zv = jnp.zeros((L,), jnp.float32)

    def zb(i, carry):
        for j in range(D // L):
            acc[i, pl.ds(j * L, L)] = zv
        return carry

    lax.fori_loop(0, TRASH + 1, zb, 0)

    pltpu.sync_copy(cnt_hbm.at[pl.ds(w * L, L)], cbuf)
    total = cbuf[...][0]
    nch = total // EK

    def decode(ci, sidx):
        for k in range(EK // L):
            rv = recbig[pl.ds(ci * EK + k * L, L)]
            sidx[pl.ds(k * L, L)] = rv & 0x3FFF

    def accumulate(ci, rows):
        for k in range(EK // L):
            dlv = recbig[pl.ds(ci * EK + k * L, L)] >> 14
            for r in range(L):
                li = dlv[r]
                for j in range(D // L):
                    plsc.addupdate(acc.at[li, pl.ds(j * L, L)],
                                   rows[k * L + r, pl.ds(j * L, L)])

    @pl.when(total <= RECCAP)
    def _fast():
        nrg = (total + 2047) // 2048

        def pre(i, carry):
            pltpu.sync_copy(
                rec_hbm.at[pl.ds(pl.multiple_of(w * CAPW + i * 2048, 8), 2048)],
                recbig.at[pl.ds(i * 2048, 2048)])
            return carry

        lax.fori_loop(0, nrg, pre, 0)

        @pl.when(nch > 0)
        def _():
            decode(0, sidxA)
            pltpu.async_copy(h_hbm.at[sidxA], rowsA, semA)

            def chunk(ci, carry):
                @pl.when(lax.rem(ci, 2) == 0)
                def _():
                    @pl.when(ci + 1 < nch)
                    def _():
                        decode(ci + 1, sidxB)
                        pltpu.async_copy(h_hbm.at[sidxB], rowsB, semB)
                    pltpu.make_async_copy(h_hbm.at[sidxA], rowsA, semA).wait()
                    accumulate(ci, rowsA)

                @pl.when(lax.rem(ci, 2) == 1)
                def _():
                    @pl.when(ci + 1 < nch)
                    def _():
                        decode(ci + 1, sidxA)
                        pltpu.async_copy(h_hbm.at[sidxA], rowsA, semA)
                    pltpu.make_async_copy(h_hbm.at[sidxB], rowsB, semB).wait()
                    accumulate(ci, rowsB)

                return carry

            lax.fori_loop(0, nch, chunk, 0)

    @pl.when(total > RECCAP)
    def _slow():
        def chunk(ci, carry):
            pltpu.sync_copy(
                rec_hbm.at[pl.ds(pl.multiple_of(w * CAPW + ci * EK, 8), EK)],
                recbig.at[pl.ds(0, EK)])
            decode(0, sidxA)
            pltpu.async_copy(h_hbm.at[sidxA], rowsA, semA).wait()
            accumulate(0, rowsA)
            return carry

        lax.fori_loop(0, nch, chunk, 0)

    @pl.when(w < NW - 1)
    def _():
        pltpu.sync_copy(acc.at[pl.ds(0, SLAB)], out_hbm.at[pl.ds(w * SLAB, SLAB)])

    @pl.when(w == NW - 1)
    def _():
        pltpu.sync_copy(acc.at[pl.ds(0, SLAB_LAST)], out_hbm.at[pl.ds((NW - 1) * SLAB, SLAB_LAST)])


@jax.jit
def _edge_agg(h, rec, cnt):
    return pl.kernel(
        _edge_agg_body,
        out_type=jax.ShapeDtypeStruct((N, D), jnp.float32),
        mesh=_MESH,
        scratch_types=[
            pltpu.VMEM((RECCAP,), jnp.int32),
            pltpu.VMEM((EK,), jnp.int32),
            pltpu.VMEM((EK,), jnp.int32),
            pltpu.VMEM((EK, D), jnp.float32),
            pltpu.VMEM((EK, D), jnp.float32),
            pltpu.VMEM((TRASH + 1, D), jnp.float32),
            pltpu.VMEM((L,), jnp.int32),
            pltpu.SemaphoreType.DMA,
            pltpu.SemaphoreType.DMA,
        ],
        compiler_params=pltpu.CompilerParams(needs_layout_passes=False),
    )(h, rec, cnt)


def _pool_body(h_hbm, batch_hbm, pmax_hbm, psum_hbm, accm, accs, rowbuf, bbufv, sem):
    w = _worker_id()
    base = w * PSLAB

    ninf = jnp.full((L,), -jnp.inf, jnp.float32)
    zv = jnp.zeros((L,), jnp.float32)

    def init(i, carry):
        for j in range(D // L):
            accm[i, pl.ds(j * L, L)] = ninf
            accs[i, pl.ds(j * L, L)] = zv
        return carry

    lax.fori_loop(0, B, init, 0)

    @pl.when(w < NW - 1)
    def _():
        pltpu.sync_copy(batch_hbm.at[pl.ds(base, PSLAB)], bbufv)

    @pl.when(w == NW - 1)
    def _():
        pltpu.sync_copy(batch_hbm.at[pl.ds(base, PSLAB_LAST)], bbufv.at[pl.ds(0, PSLAB_LAST)])

    nchunk = jnp.where(w == NW - 1, PSLAB_LAST // L, PSLAB // L)

    def chunk(ci, carry):
        pltpu.sync_copy(h_hbm.at[pl.ds(base + ci * L, L)], rowbuf)
        bv = bbufv[pl.ds(ci * L, L)]
        for r in range(L):
            bid = bv[r]
            for j in range(D // L):
                v = rowbuf[r, pl.ds(j * L, L)]
                m = accm[bid, pl.ds(j * L, L)]
                accm[bid, pl.ds(j * L, L)] = jnp.maximum(m, v)
                sv = accs[bid, pl.ds(j * L, L)]
                accs[bid, pl.ds(j * L, L)] = sv + v
        return carry

    lax.fori_loop(0, nchunk, chunk, 0)

    pltpu.sync_copy(accm, pmax_hbm.at[w])
    pltpu.sync_copy(accs, psum_hbm.at[w])


@jax.jit
def _pool(h, batch):
    return pl.kernel(
        _pool_body,
        out_type=(
            jax.ShapeDtypeStruct((NW, B, D), jnp.float32),
            jax.ShapeDtypeStruct((NW, B, D), jnp.float32),
        ),
        mesh=_MESH,
        scratch_types=[
            pltpu.VMEM((B, D), jnp.float32),
            pltpu.VMEM((B, D), jnp.float32),
            pltpu.VMEM((L, D), jnp.float32),
            pltpu.VMEM((PSLAB,), jnp.int32),
            pltpu.SemaphoreType.DMA,
        ],
    )(h, batch)


RBLK = 1000


def _dense_body(agg_ref, h_ref, Wrel_ref, brel_ref, Wroot_ref, p_ref, out_ref):
    agg = agg_ref[...]
    h = h_ref[...]
    g = (
        jnp.dot(agg, Wrel_ref[...], preferred_element_type=jnp.float32)
        + brel_ref[...]
        + jnp.dot(h, Wroot_ref[...], preferred_element_type=jnp.float32)
    )
    g = jnp.where(g >= 0.0, g, 0.01 * g)
    p = p_ref[...]
    pnorm = jnp.sqrt(jnp.sum(p * p))
    score = jnp.dot(g, p, preferred_element_type=jnp.float32) / (pnorm + 1e-16)
    out_ref[...] = g * jnp.tanh(score)


@jax.jit
def _dense(agg, h, Wrel, brel2d, Wroot, p2d):
    return pl.pallas_call(
        _dense_body,
        grid=(N // RBLK,),
        in_specs=[
            pl.BlockSpec((RBLK, D), lambda i: (i, 0)),
            pl.BlockSpec((RBLK, D), lambda i: (i, 0)),
            pl.BlockSpec((D, D), lambda i: (0, 0)),
            pl.BlockSpec((1, D), lambda i: (0, 0)),
            pl.BlockSpec((D, D), lambda i: (0, 0)),
            pl.BlockSpec((D, 1), lambda i: (0, 0)),
        ],
        out_specs=pl.BlockSpec((RBLK, D), lambda i: (i, 0)),
        out_shape=jax.ShapeDtypeStruct((N, D), jnp.float32),
    )(agg, h, Wrel, brel2d, Wroot, p2d)


def _final_body(
    pm1, ps1, pm2, ps2, pm3, ps3, batch_ref,
    Wl1_ref, bl1_ref, g1_ref, be1_ref,
    Wl2_ref, bl2_ref, g2_ref, be2_ref,
    Wl3_ref, bl3_ref, out_ref,
    m1, s1, m2, s2, m3, s3,
):
    i = pl.program_id(0)

    @pl.when(i == 0)
    def _():
        m1[...] = pm1[0]
        s1[...] = ps1[0]
        m2[...] = pm2[0]
        s2[...] = ps2[0]
        m3[...] = pm3[0]
        s3[...] = ps3[0]

    @pl.when(i > 0)
    def _():
        m1[...] = jnp.maximum(m1[...], pm1[0])
        s1[...] = s1[...] + ps1[0]
        m2[...] = jnp.maximum(m2[...], pm2[0])
        s2[...] = s2[...] + ps2[0]
        m3[...] = jnp.maximum(m3[...], pm3[0])
        s3[...] = s3[...] + ps3[0]

    @pl.when(i == NW - 1)
    def _():
        rows = lax.broadcasted_iota(jnp.int32, (B, 1), 0)
        cnt = jnp.zeros((B, 1), jnp.float32)
        for cidx in range(10):
            bc = batch_ref[0:1, cidx * 1000:(cidx + 1) * 1000]
            eq = (bc == rows).astype(jnp.float32)
            cnt = cnt + jnp.sum(eq, axis=1, keepdims=True)
        denom = jnp.maximum(cnt, 1.0)

        def xcat(m, s):
            gmp = jnp.where(jnp.isfinite(m[...]), m[...], 0.0)
            gap = s[...] / denom
            return gmp, gap

        gmp1, gap1 = xcat(m1, s1)
        gmp2, gap2 = xcat(m2, s2)
        gmp3, gap3 = xcat(m3, s3)
        z = jnp.concatenate(
            [gmp1 + gmp2 + gmp3, gap1 + gap2 + gap3], axis=1
        )

        def lrelu(v):
            return jnp.where(v >= 0.0, v, 0.01 * v)

        def bn(v, gm, bt):
            mu = jnp.mean(v, axis=0, keepdims=True)
            var = jnp.mean((v - mu) ** 2, axis=0, keepdims=True)
            return (v - mu) / jnp.sqrt(var + 1e-5) * gm + bt

        z = jnp.dot(z, Wl1_ref[...], preferred_element_type=jnp.float32) + bl1_ref[...]
        z = bn(lrelu(z), g1_ref[...], be1_ref[...])
        z = jnp.dot(z, Wl2_ref[...], preferred_element_type=jnp.float32) + bl2_ref[...]
        z = bn(lrelu(z), g2_ref[...], be2_ref[...])
        z = jnp.dot(z, Wl3_ref[...], preferred_element_type=jnp.float32) + bl3_ref[...]
        out_ref[...] = z


@jax.jit
def _final(pools, batch2d, Wl1, bl1, g1, be1, Wl2, bl2, g2, be2, Wl3, bl3):
    pm1, ps1, pm2, ps2, pm3, ps3 = pools
    part_spec = pl.BlockSpec((1, B, D), lambda i: (i, 0, 0))
    full = lambda a: pl.BlockSpec(a.shape, lambda i: tuple(0 for _ in a.shape))
    return pl.pallas_call(
        _final_body,
        grid=(NW,),
        in_specs=[part_spec] * 6 + [
            full(batch2d), full(Wl1), full(bl1), full(g1), full(be1),
            full(Wl2), full(bl2), full(g2), full(be2), full(Wl3), full(bl3),
        ],
        out_specs=pl.BlockSpec((B, 1), lambda i: (0, 0)),
        out_shape=jax.ShapeDtypeStruct((B, 1), jnp.float32),
        scratch_shapes=[pltpu.VMEM((B, D), jnp.float32)] * 6,
        compiler_params=pltpu.CompilerParams(
            dimension_semantics=("arbitrary",),
        ),
    )(pm1, ps1, pm2, ps2, pm3, ps3, batch2d, Wl1, bl1, g1, be1,
      Wl2, bl2, g2, be2, Wl3, bl3)


def kernel(x, edge_index, batch, emb, Wrel1, brel1, Wroot1, p1, Wrel2, brel2,
           Wroot2, p2, Wrel3, brel3, Wroot3, p3, Wl1, bl1, g1, be1, Wl2, bl2,
           g2, be2, Wl3, bl3):
    xv = x[:, 0]
    src = edge_index[0]
    dst = edge_index[1]
    batch2d = batch[None, :]

    h = _emb_gather(xv, emb)
    rec, cnt = _bucket(src, dst)

    layer_params = (
        (Wrel1, brel1, Wroot1, p1),
        (Wrel2, brel2, Wroot2, p2),
        (Wrel3, brel3, Wroot3, p3),
    )
    pools = []
    for Wrel, brel, Wroot, p in layer_params:
        agg = _edge_agg(h, rec, cnt)
        h = _dense(agg, h, Wrel, brel[None, :], Wroot, p[:, None])
        pm, ps = _pool(h, batch)
        pools.extend([pm, ps])

    out = _final(
        tuple(pools), batch2d,
        Wl1, bl1[None, :], g1[None, :], be1[None, :],
        Wl2, bl2[None, :], g2[None, :], be2[None, :],
        Wl3, bl3[None, :],
    )
    return out[:, 0]


# speedup vs baseline: 1.0386x; 1.0386x over previous
import functools

import jax
import jax.numpy as jnp
from jax import lax
from jax.experimental import pallas as pl
from jax.experimental.pallas import tpu as pltpu
from jax.experimental.pallas import tpu_sc as plsc

N = 10000
E = 320000
B = 256
LIB = 1000
D = 128

NC = 2
NS = 16
NW = NC * NS
L = 16

SLAB = 312
SLAB_LAST = SLAB + (N - SLAB * NW)
PSLAB = 320
PSLAB_LAST = N - PSLAB * (NW - 1)
EPW = E // NW
EK = 80
ESTEPS = EPW // EK
RPW = 624
RPW_LAST = N - RPW * (NS - 1)

_MESH = plsc.VectorSubcoreMesh(core_axis_name="c", subcore_axis_name="s")


def _worker_id():
    return lax.axis_index("s") * NC + lax.axis_index("c")


def _emb_gather_body(xv_hbm, emb_hbm, out_hbm, idx_v, rows_v, idx2_v, rows2_v, sem):
    w = _worker_id()
    base = w * SLAB
    for i in range(3):
        off = base + i * 104
        pltpu.sync_copy(xv_hbm.at[pl.ds(off, 104)], idx_v)
        pltpu.async_copy(emb_hbm.at[idx_v], rows_v, sem).wait()
        pltpu.sync_copy(rows_v, out_hbm.at[pl.ds(off, 104)])

    @pl.when(w == NW - 1)
    def _():
        off = base + SLAB
        pltpu.sync_copy(xv_hbm.at[pl.ds(off, 16)], idx2_v)
        pltpu.async_copy(emb_hbm.at[idx2_v], rows2_v, sem).wait()
        pltpu.sync_copy(rows2_v, out_hbm.at[pl.ds(off, 16)])


@jax.jit
def _emb_gather(xv, emb):
    return pl.kernel(
        _emb_gather_body,
        out_type=jax.ShapeDtypeStruct((N, D), jnp.float32),
        mesh=_MESH,
        scratch_types=[
            pltpu.VMEM((104,), jnp.int32),
            pltpu.VMEM((104, D), jnp.float32),
            pltpu.VMEM((16,), jnp.int32),
            pltpu.VMEM((16, D), jnp.float32),
            pltpu.SemaphoreType.DMA,
        ],
    )(xv, emb)


ECHUNK = 1600
TRASH = 335
TRASH_REC = TRASH << 14
CAPW = E + EK
VBUF = 2176
FLUSH = 2048


def _prefix16(m):
    iota = lax.iota(jnp.int32, L)
    for k in (1, 2, 4, 8):
        g = m.at[jnp.maximum(iota - k, 0)].get(mode="promise_in_bounds")
        m = m + jnp.where(iota >= k, g, 0)
    return m


def _bucket_body(src_hbm, dst_hbm, rec_hbm, cnt_hbm, svbuf, dvbuf, vbuf, cbuf, sem):
    w = _worker_id()
    low = w * SLAB
    hi = low + jnp.where(w == NW - 1, SLAB_LAST, SLAB)
    full_mask = jnp.ones((L,), jnp.bool_)
    tvec = jnp.full((L,), TRASH_REC, jnp.int32)

    def outer(oi, carry):
        pltpu.sync_copy(src_hbm.at[pl.ds(oi * ECHUNK, ECHUNK)], svbuf)
        pltpu.sync_copy(dst_hbm.at[pl.ds(oi * ECHUNK, ECHUNK)], dvbuf)

        def inner(gk, c2):
            vcur, gcur = c2
            recs, masks, cms = [], [], []
            for u in range(4):
                sv = svbuf[pl.ds((gk * 4 + u) * L, L)]
                dv = dvbuf[pl.ds((gk * 4 + u) * L, L)]
                mask = (dv >= low) & (dv < hi)
                rec = sv | ((dv - low) << 14)
                cm = _prefix16(jnp.where(mask, 1, 0))
                recs.append(rec)
                masks.append(mask)
                cms.append(cm)
            base = vcur
            for u in range(4):
                plsc.store_scatter(vbuf, [base + cms[u] - 1], recs[u], mask=masks[u])
                base = base + cms[u][L - 1]
            vcur = base

            def do_flush(args):
                vc, gc = args
                pltpu.sync_copy(vbuf.at[pl.ds(0, FLUSH)], rec_hbm.at[pl.ds(pl.multiple_of(w * CAPW + gc, 2048), FLUSH)])
                for t in range(4):
                    tail = vbuf[pl.ds(FLUSH + t * L, L)]
                    vbuf[pl.ds(t * L, L)] = tail
                return (vc - FLUSH, gc + FLUSH)

            return lax.cond(vcur >= FLUSH, do_flush, lambda a: a, (vcur, gcur))

        return lax.fori_loop(0, ECHUNK // L // 4, inner, carry)

    vcur, gcur = lax.fori_loop(0, E // ECHUNK, outer, (jnp.int32(0), jnp.int32(0)))

    plsc.store_scatter(vbuf, [vcur + lax.iota(jnp.int32, L)], tvec, mask=full_mask)
    vcur = ((vcur + L - 1) // L) * L

    def pad_more(args):
        vc, gc = args
        plsc.store_scatter(vbuf, [vc + lax.iota(jnp.int32, L)], tvec, mask=full_mask)
        return (vc + L, gc)

    def pad_cond(args):
        vc, gc = args
        return ((gc + vc) % EK) != 0

    vcur, gcur = lax.while_loop(pad_cond, pad_more, (vcur, gcur))

    def fl(i, args):
        vc, gc = args
        pltpu.sync_copy(vbuf.at[pl.ds(i * L, L)], rec_hbm.at[pl.ds(pl.multiple_of(w * CAPW + gc + i * L, L), L)])
        return (vc, gc)

    lax.fori_loop(0, vcur // L, fl, (vcur, gcur))
    total = gcur + vcur

    cbuf[...] = jnp.where(lax.iota(jnp.int32, L) == 0, total, 0)
    pltpu.sync_copy(cbuf, cnt_hbm.at[pl.ds(w * L, L)])


@jax.jit
def _bucket(src, dst):
    return pl.kernel(
        _bucket_body,
        out_type=(
            jax.ShapeDtypeStruct((NW * CAPW,), jnp.int32),
            jax.ShapeDtypeStruct((NW * L,), jnp.int32),
        ),
        mesh=_MESH,
        scratch_types=[
            pltpu.VMEM((ECHUNK,), jnp.int32),
            pltpu.VMEM((ECHUNK,), jnp.int32),
            pltpu.VMEM((VBUF,), jnp.int32),
            pltpu.VMEM((L,), jnp.int32),
            pltpu.SemaphoreType.DMA,
        ],
        compiler_params=pltpu.CompilerParams(needs_layout_passes=False),
    )(src, dst)


RECCAP = 16384


def _edge_agg_body(h_hbm, rec_hbm, cnt_hbm, out_hbm, recbig, sidxA, sidxB,
                   rowsA, rowsB, acc, cbuf, semA, semB):
    w = _worker_id()
    zv = jnp.zeros((L,), jnp.float32)

    def zb(i, carry):
        for j in range(D // L):
            acc[i, pl.ds(j * L, L)] = zv
        return carry

    lax.fori_loop(0, TRASH + 1, zb, 0)

    pltpu.sync_copy(cnt_hbm.at[pl.ds(w * L, L)], cbuf)
    total = cbuf[...][0]
    nch = total // EK

    def decode(ci, sidx):
        for k in range(EK // L):
            rv = recbig[pl.ds(ci * EK + k * L, L)]
            sidx[pl.ds(k * L, L)] = rv & 0x3FFF

    def accumulate(ci, rows):
        for k in range(EK // L):
            dlv = recbig[pl.ds(ci * EK + k * L, L)] >> 14
            for r in range(L):
                li = dlv[r]
                for j in range(D // L):
                    plsc.addupdate(acc.at[li, pl.ds(j * L, L)],
                                   rows[k * L + r, pl.ds(j * L, L)])

    @pl.when(total <= RECCAP)
    def _fast():
        nrg = (total + 2047) // 2048

        def pre(i, carry):
            pltpu.sync_copy(
                rec_hbm.at[pl.ds(pl.multiple_of(w * CAPW + i * 2048, 8), 2048)],
                recbig.at[pl.ds(i * 2048, 2048)])
            return carry

        lax.fori_loop(0, nrg, pre, 0)

        @pl.when(nch > 0)
        def _():
            decode(0, sidxA)
            pltpu.async_copy(h_hbm.at[sidxA], rowsA, semA)

            def chunk(ci, carry):
                @pl.when(lax.rem(ci, 2) == 0)
                def _():
                    @pl.when(ci + 1 < nch)
                    def _():
                        decode(ci + 1, sidxB)
                        pltpu.async_copy(h_hbm.at[sidxB], rowsB, semB)
                    pltpu.make_async_copy(h_hbm.at[sidxA], rowsA, semA).wait()
                    accumulate(ci, rowsA)

                @pl.when(lax.rem(ci, 2) == 1)
                def _():
                    @pl.when(ci + 1 < nch)
                    def _():
                        decode(ci + 1, sidxA)
                        pltpu.async_copy(h_hbm.at[sidxA], rowsA, semA)
                    pltpu.make_async_copy(h_hbm.at[sidxB], rowsB, semB).wait()
                    accumulate(ci, rowsB)

                return carry

            lax.fori_loop(0, nch, chunk, 0)

    @pl.when(total > RECCAP)
    def _slow():
        def chunk(ci, carry):
            pltpu.sync_copy(
                rec_hbm.at[pl.ds(pl.multiple_of(w * CAPW + ci * EK, 8), EK)],
                recbig.at[pl.ds(0, EK)])
            decode(0, sidxA)
            pltpu.async_copy(h_hbm.at[sidxA], rowsA, semA).wait()
            accumulate(0, rowsA)
            return carry

        lax.fori_loop(0, nch, chunk, 0)

    @pl.when(w < NW - 1)
    def _():
        pltpu.sync_copy(acc.at[pl.ds(0, SLAB)], out_hbm.at[pl.ds(w * SLAB, SLAB)])

    @pl.when(w == NW - 1)
    def _():
        pltpu.sync_copy(acc.at[pl.ds(0, SLAB_LAST)], out_hbm.at[pl.ds((NW - 1) * SLAB, SLAB_LAST)])


@jax.jit
def _edge_agg(h, rec, cnt):
    return pl.kernel(
        _edge_agg_body,
        out_type=jax.ShapeDtypeStruct((N, D), jnp.float32),
        mesh=_MESH,
        scratch_types=[
            pltpu.VMEM((RECCAP,), jnp.int32),
            pltpu.VMEM((EK,), jnp.int32),
            pltpu.VMEM((EK,), jnp.int32),
            pltpu.VMEM((EK, D), jnp.float32),
            pltpu.VMEM((EK, D), jnp.float32),
            pltpu.VMEM((TRASH + 1, D), jnp.float32),
            pltpu.VMEM((L,), jnp.int32),
            pltpu.SemaphoreType.DMA,
            pltpu.SemaphoreType.DMA,
        ],
        compiler_params=pltpu.CompilerParams(needs_layout_passes=False),
    )(h, rec, cnt)


def _pool_body(h_hbm, batch_hbm, pmax_hbm, psum_hbm, accm, accs, rowbuf, bbufv, sem):
    w = _worker_id()
    base = w * PSLAB

    ninf = jnp.full((L,), -jnp.inf, jnp.float32)
    zv = jnp.zeros((L,), jnp.float32)

    def init(i, carry):
        for j in range(D // L):
            accm[i, pl.ds(j * L, L)] = ninf
            accs[i, pl.ds(j * L, L)] = zv
        return carry

    lax.fori_loop(0, B, init, 0)

    @pl.when(w < NW - 1)
    def _():
        pltpu.sync_copy(batch_hbm.at[pl.ds(base, PSLAB)], bbufv)

    @pl.when(w == NW - 1)
    def _():
        pltpu.sync_copy(batch_hbm.at[pl.ds(base, PSLAB_LAST)], bbufv.at[pl.ds(0, PSLAB_LAST)])

    nchunk = jnp.where(w == NW - 1, PSLAB_LAST // L, PSLAB // L)

    def chunk(ci, carry):
        pltpu.sync_copy(h_hbm.at[pl.ds(base + ci * L, L)], rowbuf)
        bv = bbufv[pl.ds(ci * L, L)]
        for r in range(L):
            bid = bv[r]
            for j in range(D // L):
                v = rowbuf[r, pl.ds(j * L, L)]
                m = accm[bid, pl.ds(j * L, L)]
                accm[bid, pl.ds(j * L, L)] = jnp.maximum(m, v)
                sv = accs[bid, pl.ds(j * L, L)]
                accs[bid, pl.ds(j * L, L)] = sv + v
        return carry

    lax.fori_loop(0, nchunk, chunk, 0)

    pltpu.sync_copy(accm, pmax_hbm.at[w])
    pltpu.sync_copy(accs, psum_hbm.at[w])


@jax.jit
def _pool(h, batch):
    return pl.kernel(
        _pool_body,
        out_type=(
            jax.ShapeDtypeStruct((NW, B, D), jnp.float32),
            jax.ShapeDtypeStruct((NW, B, D), jnp.float32),
        ),
        mesh=_MESH,
        scratch_types=[
            pltpu.VMEM((B, D), jnp.float32),
            pltpu.VMEM((B, D), jnp.float32),
            pltpu.VMEM((L, D), jnp.float32),
            pltpu.VMEM((PSLAB,), jnp.int32),
            pltpu.SemaphoreType.DMA,
        ],
    )(h, batch)


RBLK = 1000


def _dense_body(agg_ref, h_ref, Wrel_ref, brel_ref, Wroot_ref, p_ref, out_ref):
    agg = agg_ref[...]
    h = h_ref[...]
    g = (
        jnp.dot(agg, Wrel_ref[...], preferred_element_type=jnp.float32)
        + brel_ref[...]
        + jnp.dot(h, Wroot_ref[...], preferred_element_type=jnp.float32)
    )
    g = jnp.where(g >= 0.0, g, 0.01 * g)
    p = p_ref[...]
    pnorm = jnp.sqrt(jnp.sum(p * p))
    score = jnp.dot(g, p, preferred_element_type=jnp.float32) / (pnorm + 1e-16)
    out_ref[...] = g * jnp.tanh(score)


@jax.jit
def _dense(agg, h, Wrel, brel2d, Wroot, p2d):
    return pl.pallas_call(
        _dense_body,
        grid=(N // RBLK,),
        in_specs=[
            pl.BlockSpec((RBLK, D), lambda i: (i, 0)),
            pl.BlockSpec((RBLK, D), lambda i: (i, 0)),
            pl.BlockSpec((D, D), lambda i: (0, 0)),
            pl.BlockSpec((1, D), lambda i: (0, 0)),
            pl.BlockSpec((D, D), lambda i: (0, 0)),
            pl.BlockSpec((D, 1), lambda i: (0, 0)),
        ],
        out_specs=pl.BlockSpec((RBLK, D), lambda i: (i, 0)),
        out_shape=jax.ShapeDtypeStruct((N, D), jnp.float32),
    )(agg, h, Wrel, brel2d, Wroot, p2d)


def _final_body(
    pm1, ps1, pm2, ps2, pm3, ps3, batch_ref,
    Wl1_ref, bl1_ref, g1_ref, be1_ref,
    Wl2_ref, bl2_ref, g2_ref, be2_ref,
    Wl3_ref, bl3_ref, out_ref,
    m1, s1, m2, s2, m3, s3,
):
    i = pl.program_id(0)

    @pl.when(i == 0)
    def _():
        m1[...] = pm1[0]
        s1[...] = ps1[0]
        m2[...] = pm2[0]
        s2[...] = ps2[0]
        m3[...] = pm3[0]
        s3[...] = ps3[0]

    @pl.when(i > 0)
    def _():
        m1[...] = jnp.maximum(m1[...], pm1[0])
        s1[...] = s1[...] + ps1[0]
        m2[...] = jnp.maximum(m2[...], pm2[0])
        s2[...] = s2[...] + ps2[0]
        m3[...] = jnp.maximum(m3[...], pm3[0])
        s3[...] = s3[...] + ps3[0]

    @pl.when(i == NW - 1)
    def _():
        rows = lax.broadcasted_iota(jnp.int32, (B, 1), 0)
        cnt = jnp.zeros((B, 1), jnp.float32)
        for cidx in range(10):
            bc = batch_ref[0:1, cidx * 1000:(cidx + 1) * 1000]
            eq = (bc == rows).astype(jnp.float32)
            cnt = cnt + jnp.sum(eq, axis=1, keepdims=True)
        denom = jnp.maximum(cnt, 1.0)

        def xcat(m, s):
            gmp = jnp.where(jnp.isfinite(m[...]), m[...], 0.0)
            gap = s[...] / denom
            return gmp, gap

        gmp1, gap1 = xcat(m1, s1)
        gmp2, gap2 = xcat(m2, s2)
        gmp3, gap3 = xcat(m3, s3)
        z = jnp.concatenate(
            [gmp1 + gmp2 + gmp3, gap1 + gap2 + gap3], axis=1
        )

        def lrelu(v):
            return jnp.where(v >= 0.0, v, 0.01 * v)

        def bn(v, gm, bt):
            mu = jnp.mean(v, axis=0, keepdims=True)
            var = jnp.mean((v - mu) ** 2, axis=0, keepdims=True)
            return (v - mu) / jnp.sqrt(var + 1e-5) * gm + bt

        z = jnp.dot(z, Wl1_ref[...], preferred_element_type=jnp.float32) + bl1_ref[...]
        z = bn(lrelu(z), g1_ref[...], be1_ref[...])
        z = jnp.dot(z, Wl2_ref[...], preferred_element_type=jnp.float32) + bl2_ref[...]
        z = bn(lrelu(z), g2_ref[...], be2_ref[...])
        z = jnp.dot(z, Wl3_ref[...], preferred_element_type=jnp.float32) + bl3_ref[...]
        out_ref[...] = z


@jax.jit
def _final(pools, batch2d, Wl1, bl1, g1, be1, Wl2, bl2, g2, be2, Wl3, bl3):
    pm1, ps1, pm2, ps2, pm3, ps3 = pools
    part_spec = pl.BlockSpec((1, B, D), lambda i: (i, 0, 0))
    full = lambda a: pl.BlockSpec(a.shape, lambda i: tuple(0 for _ in a.shape))
    return pl.pallas_call(
        _final_body,
        grid=(NW,),
        in_specs=[part_spec] * 6 + [
            full(batch2d), full(Wl1), full(bl1), full(g1), full(be1),
            full(Wl2), full(bl2), full(g2), full(be2), full(Wl3), full(bl3),
        ],
        out_specs=pl.BlockSpec((B, 1), lambda i: (0, 0)),
        out_shape=jax.ShapeDtypeStruct((B, 1), jnp.float32),
        scratch_shapes=[pltpu.VMEM((B, D), jnp.float32)] * 6,
        compiler_params=pltpu.CompilerParams(
            dimension_semantics=("arbitrary",),
        ),
    )(pm1, ps1, pm2, ps2, pm3, ps3, batch2d, Wl1, bl1, g1, be1,
      Wl2, bl2, g2, be2, Wl3, bl3)


def kernel(x, edge_index, batch, emb, Wrel1, brel1, Wroot1, p1, Wrel2, brel2,
           Wroot2, p2, Wrel3, brel3, Wroot3, p3, Wl1, bl1, g1, be1, Wl2, bl2,
           g2, be2, Wl3, bl3):
    xv = x[:, 0]
    src = edge_index[0]
    dst = edge_index[1]
    batch2d = batch[None, :]

    h = _emb_gather(xv, emb)
    rec, cnt = _bucket(src, dst)

    layer_params = (
        (Wrel1, brel1, Wroot1, p1),
        (Wrel2, brel2, Wroot2, p2),
        (Wrel3, brel3, Wroot3, p3),
    )
    pools = []
    for Wrel, brel, Wroot, p in layer_params:
        agg = _edge_agg(h, rec, cnt)
        h = _dense(agg, h, Wrel, brel[None, :], Wroot, p[:, None])
        pm, ps = _pool(h, batch)
        pools.extend([pm, ps])

    out = _final(
        tuple(pools), batch2d,
        Wl1, bl1[None, :], g1[None, :], be1[None, :],
        Wl2, bl2[None, :], g2[None, :], be2[None, :],
        Wl3, bl3[None, :],
    )
    return out[:, 0]
